# baseline (speedup 1.0000x reference)
import functools

import numpy as np
import jax
import jax.numpy as jnp
from jax import lax
from jax.experimental import pallas as pl
from jax.experimental.pallas import tpu as pltpu
from jax.experimental.pallas import tpu_sc as plsc

N = 1024
T = 96
K = 20
E = N * K
RANK_LR = 8
ROLE_DIM = 8
ROLE_ITERS = 80
SPEC_RANK = 16
SPEC_MOM = 4
S_SRC = 4
TAU = 1.0
ALPHA, BETA, GAMMA, DELTA = 0.3, 0.4, 0.2, 0.1
EPS = 1e-8

TBL_W = 128
NW = 32
CH = E // NW
RW = N // NW

_G = np.random.RandomState(0).randn(N, RANK_LR).astype(np.float32)
_Z = np.random.RandomState(1).randn(N, SPEC_RANK).astype(np.float32)
_W0 = np.random.RandomState(2).rand(N, ROLE_DIM).astype(np.float32)

_MM = jnp.bfloat16


def _k1_body(x_ref, s_ref, m_ref):
    x = x_ref[...]
    s = x.sum(axis=0) * 0.125
    mu = jnp.mean(s, axis=1, keepdims=True)
    var = jnp.mean((s - mu) ** 2, axis=1, keepdims=True)
    s = (s - mu) / (jnp.sqrt(var) + EPS)
    s_ref[...] = s
    sb = s.astype(_MM)
    c0 = lax.dot_general(sb, sb, (((1,), (1,)), ((), ())),
                         preferred_element_type=jnp.float32) / T
    c1 = lax.dot_general(sb[:, 1:], sb[:, :-1], (((1,), (1,)), ((), ())),
                         preferred_element_type=jnp.float32) / (T - 1)
    m = 0.5 * c0 + 0.25 * (c1 + c1.T)
    m_ref[...] = jnp.abs(m)


def _k1(x4, interpret=False):
    return pl.pallas_call(
        _k1_body,
        out_shape=(jax.ShapeDtypeStruct((N, T), jnp.float32),
                   jax.ShapeDtypeStruct((N, N), jnp.float32)),
        interpret=interpret,
    )(x4)


def _k2_body(m_ref, idx_ref):
    a = m_ref[...]
    row = lax.broadcasted_iota(jnp.int32, (N, N), 0)
    col = lax.broadcasted_iota(jnp.int32, (N, N), 1)
    a = jnp.where(row == col, -1e9, a)
    rbase = lax.broadcasted_iota(jnp.int32, (N, 1), 0) * N
    cols = []
    for _ in range(K):
        mx = jnp.max(a, axis=1, keepdims=True)
        amax = jnp.min(jnp.where(a >= mx, col, N), axis=1, keepdims=True)
        cols.append(rbase + amax)
        a = jnp.where(col == amax, -3e9, a)
    idx_ref[...] = jnp.concatenate(cols, axis=1)


def _k2(m0, interpret=False):
    return pl.pallas_call(
        _k2_body,
        out_shape=jax.ShapeDtypeStruct((N, K), jnp.int32),
        interpret=interpret,
    )(m0)


def _k3_body(m_ref, g_ref, a_ref):
    m = m_ref[...]
    y = jnp.dot(m.astype(_MM), g_ref[...].astype(_MM),
                preferred_element_type=jnp.float32)
    qcols = []
    for j in range(RANK_LR):
        v = y[:, j:j + 1]
        for q in qcols:
            v = v - q * jnp.sum(q * v)
        nrm = jnp.sqrt(jnp.sum(v * v))
        qcols.append(v / nrm)
    q = jnp.concatenate(qcols, axis=1)
    qtm = lax.dot_general(q, m, (((0,), (0,)), ((), ())),
                          preferred_element_type=jnp.float32)
    l = jnp.dot(q, qtm, preferred_element_type=jnp.float32)
    l = jnp.abs(l)
    mx = jnp.max(l, axis=1, keepdims=True)
    e = jnp.exp(l - mx)
    a_ref[...] = e / jnp.sum(e, axis=1, keepdims=True)


def _k3(m0, interpret=False):
    return pl.pallas_call(
        _k3_body,
        out_shape=jax.ShapeDtypeStruct((N, N), jnp.float32),
        interpret=interpret,
    )(m0, jnp.asarray(_G))


def _spec_moments(a, z):
    r = jnp.sum(a, axis=1)
    sc = 1.0 / (jnp.max(r) + EPS)
    v = z
    moms = []
    for _ in range(SPEC_MOM):
        v = jnp.dot(a, v, preferred_element_type=jnp.float32) * sc
        moms.append(jnp.reshape(jnp.sum(z * v) * (1.0 / (SPEC_RANK * N)), (1, 1)))
    return jnp.concatenate(moms, axis=1), r


def _k4_body(c_ref, m_ref, z_ref, rho_ref, den_ref, moms_sc, rs_sc):
    s = pl.program_id(0)
    z = z_ref[...]

    @pl.when(s == 0)
    def _():
        moms, _ = _spec_moments(m_ref[...], z)
        moms_sc[4:5, :] = moms

    moms, r = _spec_moments(c_ref[0], z)
    rs_sc[pl.ds(s, 1), :] = r.reshape(1, N)
    moms_sc[pl.ds(s, 1), :] = moms

    @pl.when(s == S_SRC - 1)
    def _():
        mm = moms_sc[...]
        m0m = mm[4:5, :]
        d = jnp.sqrt(jnp.sum((mm[:4, :] - m0m) ** 2, axis=1, keepdims=True))
        nd = -d / TAU
        ex = jnp.exp(nd - jnp.max(nd))
        rho = ex / jnp.sum(ex)
        rho_ref[...] = rho.reshape(1, S_SRC)
        den = jnp.sum(rs_sc[...] * rho, axis=0) + EPS
        den_ref[...] = den.reshape(N, 1)


def _k4(m0, cache, interpret=False):
    return pl.pallas_call(
        _k4_body,
        grid=(S_SRC,),
        in_specs=[
            pl.BlockSpec((1, N, N), lambda s: (s, 0, 0)),
            pl.BlockSpec((N, N), lambda s: (0, 0)),
            pl.BlockSpec((N, SPEC_RANK), lambda s: (0, 0)),
        ],
        out_specs=[
            pl.BlockSpec((1, S_SRC), lambda s: (0, 0)),
            pl.BlockSpec((N, 1), lambda s: (0, 0)),
        ],
        out_shape=(jax.ShapeDtypeStruct((1, S_SRC), jnp.float32),
                   jax.ShapeDtypeStruct((N, 1), jnp.float32)),
        scratch_shapes=[pltpu.VMEM((8, SPEC_MOM), jnp.float32),
                        pltpu.VMEM((S_SRC, N), jnp.float32)],
        interpret=interpret,
    )(cache, m0, jnp.asarray(_Z))


def _k5_body(m_ref, w0_ref, h_ref):
    m = m_ref[...]
    scale = jnp.sqrt(jnp.mean(m) / ROLE_DIM + EPS)
    w = w0_ref[...] * scale

    def it(_, w):
        num = jnp.dot(m, w, preferred_element_type=jnp.float32)
        wtw = lax.dot_general(w, w, (((0,), (0,)), ((), ())),
                              preferred_element_type=jnp.float32)
        den = jnp.dot(w, wtw, preferred_element_type=jnp.float32) + EPS
        return jnp.maximum(w * (num / den), 0.0)

    w = lax.fori_loop(0, ROLE_ITERS, it, w)
    nrm = jnp.sqrt(jnp.sum(w * w, axis=1, keepdims=True))
    h = w / (nrm + EPS)
    h_ref[...] = jnp.concatenate([h, jnp.zeros((N, 8), jnp.float32)], axis=1)


def _k5(m0, interpret=False):
    return pl.pallas_call(
        _k5_body,
        out_shape=jax.ShapeDtypeStruct((N, 16), jnp.float32),
        interpret=interpret,
    )(m0, jnp.asarray(_W0))


_NCHUNK = CH // 128


def _sc_gather_body(idxf, alr, c0, c1, c2, c3, table, vlr, vc, xj,
                    fidx_v, jidx_v, val_v, vc_v, xj_v, sem):
    wid = lax.axis_index("s") * 2 + lax.axis_index("c")
    pbase = wid * CH

    for kk in range(_NCHUNK):
        pltpu.sync_copy(idxf.at[pl.ds(pbase + kk * 128, 128)], fidx_v.at[kk])

    for v in range(CH // 16):
        r, c = divmod(v, 8)
        f = fidx_v[r, pl.ds(c * 16, 16)]
        jidx_v[r, pl.ds(c * 16, 16)] = lax.bitwise_and(f, N - 1)

    cps = []
    for kk in range(_NCHUNK):
        cps.append(pltpu.async_copy(table.at[jidx_v.at[kk]],
                                    xj_v.at[pl.ds(kk * 128, 128)], sem))
        cps.append(pltpu.async_copy(alr.at[fidx_v.at[kk]],
                                    val_v.at[pl.ds(kk * 128, 128)], sem))
        for s, cs in enumerate((c0, c1, c2, c3)):
            cps.append(pltpu.async_copy(cs.at[fidx_v.at[kk]],
                                        vc_v.at[s, pl.ds(kk * 128, 128)], sem))
    for cp in cps:
        cp.wait()

    pltpu.sync_copy(xj_v, xj.at[pl.ds(pbase, CH)])
    pltpu.sync_copy(val_v, vlr.at[pl.ds(pbase, CH)])
    for s in range(S_SRC):
        pltpu.sync_copy(vc_v.at[s], vc.at[pl.ds(s * E + pbase, CH)])


def _sc_gather(idx_flat, alr_flat, cache, table):
    mesh = plsc.VectorSubcoreMesh(core_axis_name="c", subcore_axis_name="s")
    fn = pl.kernel(
        _sc_gather_body, mesh=mesh,
        out_type=(jax.ShapeDtypeStruct((E,), jnp.float32),
                  jax.ShapeDtypeStruct((S_SRC * E,), jnp.float32),
                  jax.ShapeDtypeStruct((E, TBL_W), jnp.float32)),
        scratch_types=[
            pltpu.VMEM((_NCHUNK, 128), jnp.int32),
            pltpu.VMEM((_NCHUNK, 128), jnp.int32),
            pltpu.VMEM((CH,), jnp.float32),
            pltpu.VMEM((S_SRC, CH), jnp.float32),
            pltpu.VMEM((CH, TBL_W), jnp.float32),
            pltpu.SemaphoreType.DMA,
        ],
    )
    return fn(idx_flat, alr_flat,
              cache[0].reshape(N * N), cache[1].reshape(N * N),
              cache[2].reshape(N * N), cache[3].reshape(N * N), table)


def _k6_body(s_ref, h_ref, xj_ref, cv_ref, rho_ref, den_ref,
             w1_ref, b1_ref, w2_ref, b2_ref, sr_ref, res_ref):
    xj = xj_ref[...]
    sb = s_ref[...]
    hb = h_ref[...]
    xjs = xj[:, :, :T]
    hj = xj[:, :, T:T + ROLE_DIM]

    rv = jnp.sum(hb[:, None, :ROLE_DIM] * hj, axis=2)
    cv = cv_ref[...]
    rho = jnp.reshape(rho_ref[...], (S_SRC, 1, 1))
    num = jnp.sum(cv * rho, axis=0)
    den = den_ref[...]
    sr_ref[...] = rv * num / den

    si = sb[:, None, :]
    feats = []
    for l in range(8):
        f = jnp.sum(si[:, :, l:] * xjs[:, :, :T - l], axis=2) / (T - l)
        feats.append(f[:, :, None])
    fe = jnp.concatenate(feats, axis=2)
    fe = fe.reshape(8 * K, 8)
    h1 = jnp.maximum(jnp.dot(fe, w1_ref[...], preferred_element_type=jnp.float32)
                     + b1_ref[...], 0.0)
    res = jnp.sum(h1 * w2_ref[...], axis=1, keepdims=True) + b2_ref[...]
    res_ref[...] = res.reshape(8, K)


def _k6(s, h16, xj3, cv3, rho, den, w1, b1r, w2r, b2r, interpret=False):
    nb = N // 8
    return pl.pallas_call(
        _k6_body,
        grid=(nb,),
        in_specs=[
            pl.BlockSpec((8, T), lambda i: (i, 0)),
            pl.BlockSpec((8, 16), lambda i: (i, 0)),
            pl.BlockSpec((8, K, TBL_W), lambda i: (i, 0, 0)),
            pl.BlockSpec((S_SRC, 8, K), lambda i: (0, i, 0)),
            pl.BlockSpec((1, S_SRC), lambda i: (0, 0)),
            pl.BlockSpec((8, 1), lambda i: (i, 0)),
            pl.BlockSpec((8, 32), lambda i: (0, 0)),
            pl.BlockSpec((1, 32), lambda i: (0, 0)),
            pl.BlockSpec((1, 32), lambda i: (0, 0)),
            pl.BlockSpec((1, 1), lambda i: (0, 0)),
        ],
        out_specs=[
            pl.BlockSpec((8, K), lambda i: (i, 0)),
            pl.BlockSpec((8, K), lambda i: (i, 0)),
        ],
        out_shape=(jax.ShapeDtypeStruct((N, K), jnp.float32),
                   jax.ShapeDtypeStruct((N, K), jnp.float32)),
        interpret=interpret,
    )(s, h16, xj3, cv3, rho, den, w1, b1r, w2r, b2r)


def _k7_body(lr_ref, sr_ref, res_ref, p_ref):
    def nzs(v):
        mu = jnp.mean(v)
        sd = jnp.sqrt(jnp.mean((v - mu) ** 2))
        return (v - mu) / (sd + EPS)

    sf = (ALPHA * nzs(lr_ref[...]) + BETA * nzs(sr_ref[...])
          + DELTA * nzs(res_ref[...]))
    sig = 1.0 / (1.0 + jnp.exp(-sf))
    mx = jnp.max(sig, axis=1, keepdims=True)
    e = jnp.exp(sig - mx)
    p_ref[...] = e / jnp.sum(e, axis=1, keepdims=True)


def _k7(lr, sr, res, interpret=False):
    return pl.pallas_call(
        _k7_body,
        out_shape=jax.ShapeDtypeStruct((N, K), jnp.float32),
        interpret=interpret,
    )(lr, sr, res)


def _sc_scatter_body(idxf, pvals, zeros, out, fidx_v, buf_v, p_v, sem):
    wid = lax.axis_index("s") * 2 + lax.axis_index("c")
    pbase = wid * CH
    rbase = wid * RW
    pltpu.sync_copy(zeros, buf_v)
    pltpu.sync_copy(buf_v, out.at[pl.ds(rbase * N, RW * N)])
    for kk in range(_NCHUNK):
        pltpu.sync_copy(idxf.at[pl.ds(pbase + kk * 128, 128)], fidx_v.at[kk])
    pltpu.sync_copy(pvals.at[pl.ds(pbase, CH)], p_v)
    cps = [pltpu.async_copy(p_v.at[pl.ds(kk * 128, 128)],
                            out.at[fidx_v.at[kk]], sem)
           for kk in range(_NCHUNK)]
    for cp in cps:
        cp.wait()


def _sc_scatter(idx_flat, p_flat, zeros):
    mesh = plsc.VectorSubcoreMesh(core_axis_name="c", subcore_axis_name="s")
    fn = pl.kernel(
        _sc_scatter_body, mesh=mesh,
        out_type=jax.ShapeDtypeStruct((N * N,), jnp.float32),
        scratch_types=[
            pltpu.VMEM((_NCHUNK, 128), jnp.int32),
            pltpu.VMEM((RW * N,), jnp.float32),
            pltpu.VMEM((CH,), jnp.float32),
            pltpu.SemaphoreType.DMA,
        ],
    )
    return fn(idx_flat, p_flat, zeros)


def kernel(x_target, source_structure_cache, W1, b1, W2, b2):
    xt = x_target.transpose(0, 3, 1, 2).reshape(8, N, T)
    s, m0 = _k1(xt)
    idx = _k2(m0)
    alr = _k3(m0)
    rho, den = _k4(m0, source_structure_cache)
    h16 = _k5(m0)

    idx_flat = idx.reshape(E)
    table = jnp.concatenate(
        [s, h16, jnp.zeros((N, TBL_W - T - 16), jnp.float32)], axis=1)
    vlr, vc, xj = _sc_gather(idx_flat, alr.reshape(N * N),
                             source_structure_cache, table)

    sr, res = _k6(s, h16, xj.reshape(N, K, TBL_W),
                  vc.reshape(S_SRC, N, K), rho, den,
                  W1, b1.reshape(1, 32), W2.reshape(1, 32), b2.reshape(1, 1))
    p = _k7(vlr.reshape(N, K), sr, res)

    zeros = jnp.zeros((RW * N,), jnp.float32)
    out = _sc_scatter(idx_flat, p.reshape(E), zeros)
    return out.reshape(N, N)


# speedup vs baseline: 1.2879x; 1.2879x over previous
import functools

import numpy as np
import jax
import jax.numpy as jnp
from jax import lax
from jax.experimental import pallas as pl
from jax.experimental.pallas import tpu as pltpu
from jax.experimental.pallas import tpu_sc as plsc

N = 1024
T = 96
K = 20
E = N * K
RANK_LR = 8
ROLE_DIM = 8
ROLE_ITERS = 80
SPEC_RANK = 16
SPEC_MOM = 4
S_SRC = 4
TAU = 1.0
ALPHA, BETA, GAMMA, DELTA = 0.3, 0.4, 0.2, 0.1
EPS = 1e-8

TBL_W = 128
NW = 32
CH = E // NW
RW = N // NW

_G = np.random.RandomState(0).randn(N, RANK_LR).astype(np.float32)
_Z = np.random.RandomState(1).randn(N, SPEC_RANK).astype(np.float32)
_W0 = np.random.RandomState(2).rand(N, ROLE_DIM).astype(np.float32)

_MM = jnp.bfloat16


def _k1_body(x_ref, s_ref, m_ref):
    x = x_ref[...]
    s = x.sum(axis=0) * 0.125
    mu = jnp.mean(s, axis=1, keepdims=True)
    var = jnp.mean((s - mu) ** 2, axis=1, keepdims=True)
    s = (s - mu) / (jnp.sqrt(var) + EPS)
    s_ref[...] = s
    sb = s.astype(_MM)
    c0 = lax.dot_general(sb, sb, (((1,), (1,)), ((), ())),
                         preferred_element_type=jnp.float32) / T
    c1 = lax.dot_general(sb[:, 1:], sb[:, :-1], (((1,), (1,)), ((), ())),
                         preferred_element_type=jnp.float32) / (T - 1)
    m = 0.5 * c0 + 0.25 * (c1 + c1.T)
    m_ref[...] = jnp.abs(m)


def _k1(x4, interpret=False):
    return pl.pallas_call(
        _k1_body,
        out_shape=(jax.ShapeDtypeStruct((N, T), jnp.float32),
                   jax.ShapeDtypeStruct((N, N), jnp.float32)),
        interpret=interpret,
    )(x4)


def _k2_body(m_ref, idx_ref):
    a = m_ref[...]
    row = lax.broadcasted_iota(jnp.int32, (N, N), 0)
    col = lax.broadcasted_iota(jnp.int32, (N, N), 1)
    a = jnp.where(row == col, -1e9, a)
    rbase = lax.broadcasted_iota(jnp.int32, (N, 1), 0) * N
    cols = []
    for _ in range(K):
        mx = jnp.max(a, axis=1, keepdims=True)
        amax = jnp.min(jnp.where(a >= mx, col, N), axis=1, keepdims=True)
        cols.append(rbase + amax)
        a = jnp.where(col == amax, -3e9, a)
    idx_ref[...] = jnp.concatenate(cols, axis=1)


def _k2(m0, interpret=False):
    return pl.pallas_call(
        _k2_body,
        out_shape=jax.ShapeDtypeStruct((N, K), jnp.int32),
        interpret=interpret,
    )(m0)


def _k3_body(m_ref, g_ref, a_ref):
    m = m_ref[...]
    y = jnp.dot(m.astype(_MM), g_ref[...].astype(_MM),
                preferred_element_type=jnp.float32)
    qcols = []
    for j in range(RANK_LR):
        v = y[:, j:j + 1]
        for q in qcols:
            v = v - q * jnp.sum(q * v)
        nrm = jnp.sqrt(jnp.sum(v * v))
        qcols.append(v / nrm)
    q = jnp.concatenate(qcols, axis=1)
    qtm = lax.dot_general(q, m, (((0,), (0,)), ((), ())),
                          preferred_element_type=jnp.float32)
    l = jnp.dot(q, qtm, preferred_element_type=jnp.float32)
    l = jnp.abs(l)
    mx = jnp.max(l, axis=1, keepdims=True)
    e = jnp.exp(l - mx)
    a_ref[...] = e / jnp.sum(e, axis=1, keepdims=True)


def _k3(m0, interpret=False):
    return pl.pallas_call(
        _k3_body,
        out_shape=jax.ShapeDtypeStruct((N, N), jnp.float32),
        interpret=interpret,
    )(m0, jnp.asarray(_G))


def _spec_moments(a, z):
    r = jnp.sum(a, axis=1)
    sc = 1.0 / (jnp.max(r) + EPS)
    v = z
    moms = []
    for _ in range(SPEC_MOM):
        v = jnp.dot(a, v, preferred_element_type=jnp.float32) * sc
        moms.append(jnp.reshape(jnp.sum(z * v) * (1.0 / (SPEC_RANK * N)), (1, 1)))
    return jnp.concatenate(moms, axis=1), r


def _k4_body(c_ref, m_ref, z_ref, rho_ref, den_ref, moms_sc, rs_sc):
    s = pl.program_id(0)
    z = z_ref[...]

    @pl.when(s == 0)
    def _():
        moms, _ = _spec_moments(m_ref[...], z)
        moms_sc[4:5, :] = moms

    moms, r = _spec_moments(c_ref[0], z)
    rs_sc[pl.ds(s, 1), :] = r.reshape(1, N)
    moms_sc[pl.ds(s, 1), :] = moms

    @pl.when(s == S_SRC - 1)
    def _():
        mm = moms_sc[...]
        m0m = mm[4:5, :]
        d = jnp.sqrt(jnp.sum((mm[:4, :] - m0m) ** 2, axis=1, keepdims=True))
        nd = -d / TAU
        ex = jnp.exp(nd - jnp.max(nd))
        rho = ex / jnp.sum(ex)
        rho_ref[...] = rho.reshape(1, S_SRC)
        den = jnp.sum(rs_sc[...] * rho, axis=0) + EPS
        den_ref[...] = den.reshape(N, 1)


def _k4(m0, cache, interpret=False):
    return pl.pallas_call(
        _k4_body,
        grid=(S_SRC,),
        in_specs=[
            pl.BlockSpec((1, N, N), lambda s: (s, 0, 0)),
            pl.BlockSpec((N, N), lambda s: (0, 0)),
            pl.BlockSpec((N, SPEC_RANK), lambda s: (0, 0)),
        ],
        out_specs=[
            pl.BlockSpec((1, S_SRC), lambda s: (0, 0)),
            pl.BlockSpec((N, 1), lambda s: (0, 0)),
        ],
        out_shape=(jax.ShapeDtypeStruct((1, S_SRC), jnp.float32),
                   jax.ShapeDtypeStruct((N, 1), jnp.float32)),
        scratch_shapes=[pltpu.VMEM((8, SPEC_MOM), jnp.float32),
                        pltpu.VMEM((S_SRC, N), jnp.float32)],
        interpret=interpret,
    )(cache, m0, jnp.asarray(_Z))


def _k5_body(m_ref, w0_ref, h_ref):
    m = m_ref[...]
    scale = jnp.sqrt(jnp.mean(m) / ROLE_DIM + EPS)
    w = w0_ref[...] * scale

    def it(_, w):
        num = jnp.dot(m, w, preferred_element_type=jnp.float32)
        wtw = lax.dot_general(w, w, (((0,), (0,)), ((), ())),
                              preferred_element_type=jnp.float32)
        den = jnp.dot(w, wtw, preferred_element_type=jnp.float32) + EPS
        return jnp.maximum(w * (num / den), 0.0)

    w = lax.fori_loop(0, ROLE_ITERS, it, w)
    nrm = jnp.sqrt(jnp.sum(w * w, axis=1, keepdims=True))
    h = w / (nrm + EPS)
    h_ref[...] = jnp.concatenate([h, jnp.zeros((N, 8), jnp.float32)], axis=1)


def _k5(m0, interpret=False):
    return pl.pallas_call(
        _k5_body,
        out_shape=jax.ShapeDtypeStruct((N, 16), jnp.float32),
        interpret=interpret,
    )(m0, jnp.asarray(_W0))


_NCHUNK = CH // 128


def _sc_gather_body(idxf, alr, c0, c1, c2, c3, table, vlr, vc, xj,
                    fidx_v, jidx_v, val_v, vc_v, xj_v, sem):
    wid = lax.axis_index("s") * 2 + lax.axis_index("c")
    pbase = wid * CH

    for kk in range(_NCHUNK):
        pltpu.sync_copy(idxf.at[pl.ds(pbase + kk * 128, 128)], fidx_v.at[kk])

    for v in range(CH // 16):
        r, c = divmod(v, 8)
        f = fidx_v[r, pl.ds(c * 16, 16)]
        jidx_v[r, pl.ds(c * 16, 16)] = lax.bitwise_and(f, N - 1)

    cps = []
    for kk in range(_NCHUNK):
        cps.append(pltpu.async_copy(table.at[jidx_v.at[kk]],
                                    xj_v.at[pl.ds(kk * 128, 128)], sem))
        cps.append(pltpu.async_copy(alr.at[fidx_v.at[kk]],
                                    val_v.at[pl.ds(kk * 128, 128)], sem))
        for s, cs in enumerate((c0, c1, c2, c3)):
            cps.append(pltpu.async_copy(cs.at[fidx_v.at[kk]],
                                        vc_v.at[s, pl.ds(kk * 128, 128)], sem))
    for cp in cps:
        cp.wait()

    pltpu.sync_copy(xj_v, xj.at[pl.ds(pbase, CH)])
    pltpu.sync_copy(val_v, vlr.at[pl.ds(pbase, CH)])
    for s in range(S_SRC):
        pltpu.sync_copy(vc_v.at[s], vc.at[pl.ds(s * E + pbase, CH)])


def _sc_gather(idx_flat, alr_flat, cache, table):
    mesh = plsc.VectorSubcoreMesh(core_axis_name="c", subcore_axis_name="s")
    fn = pl.kernel(
        _sc_gather_body, mesh=mesh,
        out_type=(jax.ShapeDtypeStruct((E,), jnp.float32),
                  jax.ShapeDtypeStruct((S_SRC * E,), jnp.float32),
                  jax.ShapeDtypeStruct((E, TBL_W), jnp.float32)),
        scratch_types=[
            pltpu.VMEM((_NCHUNK, 128), jnp.int32),
            pltpu.VMEM((_NCHUNK, 128), jnp.int32),
            pltpu.VMEM((CH,), jnp.float32),
            pltpu.VMEM((S_SRC, CH), jnp.float32),
            pltpu.VMEM((CH, TBL_W), jnp.float32),
            pltpu.SemaphoreType.DMA,
        ],
    )
    return fn(idx_flat, alr_flat,
              cache[0].reshape(N * N), cache[1].reshape(N * N),
              cache[2].reshape(N * N), cache[3].reshape(N * N), table)


_RB = 64
_EB = _RB * K


def _k6_body(s_ref, h_ref, xj_ref, cv_ref, rho_ref, den_ref,
             w1_ref, b1_ref, w2_ref, b2_ref, sr_ref, res_ref):
    xj = xj_ref[...]
    xjs = xj[:, :T]
    hj = xj[:, T:T + ROLE_DIM]

    def rep(a, w):
        return jnp.broadcast_to(a[:, None, :], (_RB, K, w)).reshape(_EB, w)

    s_rep = rep(s_ref[...], T)
    h_rep = rep(h_ref[...][:, :ROLE_DIM], ROLE_DIM)
    den_rep = rep(den_ref[...], 1)

    rv = jnp.sum(h_rep * hj, axis=1, keepdims=True)
    cv = cv_ref[...]
    rho = jnp.reshape(rho_ref[...], (S_SRC, 1))
    num = jnp.sum(cv * rho, axis=0)[:, None]
    sr_ref[...] = (rv * num / den_rep).reshape(_RB, K)

    feats = []
    for l in range(8):
        f = jnp.sum(s_rep[:, l:] * xjs[:, :T - l], axis=1, keepdims=True) / (T - l)
        feats.append(f)
    fe = jnp.concatenate(feats, axis=1)
    h1 = jnp.maximum(jnp.dot(fe, w1_ref[...], preferred_element_type=jnp.float32)
                     + b1_ref[...], 0.0)
    res = jnp.sum(h1 * w2_ref[...], axis=1, keepdims=True) + b2_ref[...]
    res_ref[...] = res.reshape(_RB, K)


def _k6(s, h16, xj, cv2, rho, den, w1, b1r, w2r, b2r, interpret=False):
    nb = N // _RB
    return pl.pallas_call(
        _k6_body,
        grid=(nb,),
        in_specs=[
            pl.BlockSpec((_RB, T), lambda i: (i, 0)),
            pl.BlockSpec((_RB, 16), lambda i: (i, 0)),
            pl.BlockSpec((_EB, TBL_W), lambda i: (i, 0)),
            pl.BlockSpec((S_SRC, _EB), lambda i: (0, i)),
            pl.BlockSpec((1, S_SRC), lambda i: (0, 0)),
            pl.BlockSpec((_RB, 1), lambda i: (i, 0)),
            pl.BlockSpec((8, 32), lambda i: (0, 0)),
            pl.BlockSpec((1, 32), lambda i: (0, 0)),
            pl.BlockSpec((1, 32), lambda i: (0, 0)),
            pl.BlockSpec((1, 1), lambda i: (0, 0)),
        ],
        out_specs=[
            pl.BlockSpec((_RB, K), lambda i: (i, 0)),
            pl.BlockSpec((_RB, K), lambda i: (i, 0)),
        ],
        out_shape=(jax.ShapeDtypeStruct((N, K), jnp.float32),
                   jax.ShapeDtypeStruct((N, K), jnp.float32)),
        interpret=interpret,
    )(s, h16, xj, cv2, rho, den, w1, b1r, w2r, b2r)


def _k7_body(lr_ref, sr_ref, res_ref, p_ref):
    def nzs(v):
        mu = jnp.mean(v)
        sd = jnp.sqrt(jnp.mean((v - mu) ** 2))
        return (v - mu) / (sd + EPS)

    sf = (ALPHA * nzs(lr_ref[...]) + BETA * nzs(sr_ref[...])
          + DELTA * nzs(res_ref[...]))
    sig = 1.0 / (1.0 + jnp.exp(-sf))
    mx = jnp.max(sig, axis=1, keepdims=True)
    e = jnp.exp(sig - mx)
    p_ref[...] = e / jnp.sum(e, axis=1, keepdims=True)


def _k7(lr, sr, res, interpret=False):
    return pl.pallas_call(
        _k7_body,
        out_shape=jax.ShapeDtypeStruct((N, K), jnp.float32),
        interpret=interpret,
    )(lr, sr, res)


def _sc_scatter_body(idxf, pvals, zeros, out, fidx_v, buf_v, p_v, sem):
    wid = lax.axis_index("s") * 2 + lax.axis_index("c")
    pbase = wid * CH
    rbase = wid * RW
    pltpu.sync_copy(zeros, buf_v)
    pltpu.sync_copy(buf_v, out.at[pl.ds(rbase * N, RW * N)])
    for kk in range(_NCHUNK):
        pltpu.sync_copy(idxf.at[pl.ds(pbase + kk * 128, 128)], fidx_v.at[kk])
    pltpu.sync_copy(pvals.at[pl.ds(pbase, CH)], p_v)
    cps = [pltpu.async_copy(p_v.at[pl.ds(kk * 128, 128)],
                            out.at[fidx_v.at[kk]], sem)
           for kk in range(_NCHUNK)]
    for cp in cps:
        cp.wait()


def _sc_scatter(idx_flat, p_flat, zeros):
    mesh = plsc.VectorSubcoreMesh(core_axis_name="c", subcore_axis_name="s")
    fn = pl.kernel(
        _sc_scatter_body, mesh=mesh,
        out_type=jax.ShapeDtypeStruct((N * N,), jnp.float32),
        scratch_types=[
            pltpu.VMEM((_NCHUNK, 128), jnp.int32),
            pltpu.VMEM((RW * N,), jnp.float32),
            pltpu.VMEM((CH,), jnp.float32),
            pltpu.SemaphoreType.DMA,
        ],
    )
    return fn(idx_flat, p_flat, zeros)


def kernel(x_target, source_structure_cache, W1, b1, W2, b2):
    xt = x_target.transpose(0, 3, 1, 2).reshape(8, N, T)
    s, m0 = _k1(xt)
    idx = _k2(m0)
    alr = _k3(m0)
    rho, den = _k4(m0, source_structure_cache)
    h16 = _k5(m0)

    idx_flat = idx.reshape(E)
    table = jnp.concatenate(
        [s, h16, jnp.zeros((N, TBL_W - T - 16), jnp.float32)], axis=1)
    vlr, vc, xj = _sc_gather(idx_flat, alr.reshape(N * N),
                             source_structure_cache, table)

    sr, res = _k6(s, h16, xj, vc.reshape(S_SRC, E), rho, den,
                  W1, b1.reshape(1, 32), W2.reshape(1, 32), b2.reshape(1, 1))
    p = _k7(vlr.reshape(N, K), sr, res)

    zeros = jnp.zeros((RW * N,), jnp.float32)
    out = _sc_scatter(idx_flat, p.reshape(E), zeros)
    return out.reshape(N, N)


# speedup vs baseline: 1.3413x; 1.0415x over previous
import functools

import numpy as np
import jax
import jax.numpy as jnp
from jax import lax
from jax.experimental import pallas as pl
from jax.experimental.pallas import tpu as pltpu
from jax.experimental.pallas import tpu_sc as plsc

N = 1024
T = 96
K = 20
E = N * K
RANK_LR = 8
ROLE_DIM = 8
ROLE_ITERS = 80
SPEC_RANK = 16
SPEC_MOM = 4
S_SRC = 4
TAU = 1.0
ALPHA, BETA, GAMMA, DELTA = 0.3, 0.4, 0.2, 0.1
EPS = 1e-8

TBL_W = 128
NW = 32
CH = E // NW
RW = N // NW

_G = np.random.RandomState(0).randn(N, RANK_LR).astype(np.float32)
_Z = np.random.RandomState(1).randn(N, SPEC_RANK).astype(np.float32)
_W0 = np.random.RandomState(2).rand(N, ROLE_DIM).astype(np.float32)

_MM = jnp.bfloat16


def _k1_body(x_ref, s_ref, m_ref):
    x = x_ref[...]
    s = x.sum(axis=0) * 0.125
    mu = jnp.mean(s, axis=1, keepdims=True)
    var = jnp.mean((s - mu) ** 2, axis=1, keepdims=True)
    s = (s - mu) / (jnp.sqrt(var) + EPS)
    s_ref[...] = s
    sb = s.astype(_MM)
    c0 = lax.dot_general(sb, sb, (((1,), (1,)), ((), ())),
                         preferred_element_type=jnp.float32) / T
    c1 = lax.dot_general(sb[:, 1:], sb[:, :-1], (((1,), (1,)), ((), ())),
                         preferred_element_type=jnp.float32) / (T - 1)
    m = 0.5 * c0 + 0.25 * (c1 + c1.T)
    m_ref[...] = jnp.abs(m)


def _k1(x4, interpret=False):
    return pl.pallas_call(
        _k1_body,
        out_shape=(jax.ShapeDtypeStruct((N, T), jnp.float32),
                   jax.ShapeDtypeStruct((N, N), jnp.float32)),
        interpret=interpret,
    )(x4)


def _k2_body(m_ref, idx_ref):
    a = m_ref[...]
    row = lax.broadcasted_iota(jnp.int32, (N, N), 0)
    col = lax.broadcasted_iota(jnp.int32, (N, N), 1)
    a = jnp.where(row == col, -1e9, a)
    rbase = lax.broadcasted_iota(jnp.int32, (N, 1), 0) * N
    cols = []
    for _ in range(K):
        mx = jnp.max(a, axis=1, keepdims=True)
        amax = jnp.min(jnp.where(a >= mx, col, N), axis=1, keepdims=True)
        cols.append(rbase + amax)
        a = jnp.where(col == amax, -3e9, a)
    idx_ref[...] = jnp.concatenate(cols, axis=1)


def _k2(m0, interpret=False):
    return pl.pallas_call(
        _k2_body,
        out_shape=jax.ShapeDtypeStruct((N, K), jnp.int32),
        interpret=interpret,
    )(m0)


def _k3_body(m_ref, g_ref, a_ref):
    m = m_ref[...]
    y = jnp.dot(m.astype(_MM), g_ref[...].astype(_MM),
                preferred_element_type=jnp.float32)
    qcols = []
    for j in range(RANK_LR):
        v = y[:, j:j + 1]
        for q in qcols:
            v = v - q * jnp.sum(q * v)
        nrm = jnp.sqrt(jnp.sum(v * v))
        qcols.append(v / nrm)
    q = jnp.concatenate(qcols, axis=1)
    qtm = lax.dot_general(q, m, (((0,), (0,)), ((), ())),
                          preferred_element_type=jnp.float32)
    l = jnp.dot(q, qtm, preferred_element_type=jnp.float32)
    l = jnp.abs(l)
    mx = jnp.max(l, axis=1, keepdims=True)
    e = jnp.exp(l - mx)
    a_ref[...] = e / jnp.sum(e, axis=1, keepdims=True)


def _k3(m0, interpret=False):
    return pl.pallas_call(
        _k3_body,
        out_shape=jax.ShapeDtypeStruct((N, N), jnp.float32),
        interpret=interpret,
    )(m0, jnp.asarray(_G))


def _spec_moments(a, z):
    r = jnp.sum(a, axis=1)
    sc = 1.0 / (jnp.max(r) + EPS)
    v = z
    moms = []
    for _ in range(SPEC_MOM):
        v = jnp.dot(a, v, preferred_element_type=jnp.float32) * sc
        moms.append(jnp.reshape(jnp.sum(z * v) * (1.0 / (SPEC_RANK * N)), (1, 1)))
    return jnp.concatenate(moms, axis=1), r


def _k4_body(c_ref, m_ref, z_ref, rho_ref, den_ref, moms_sc, rs_sc):
    s = pl.program_id(0)
    z = z_ref[...]

    @pl.when(s == 0)
    def _():
        moms, _ = _spec_moments(m_ref[...], z)
        moms_sc[4:5, :] = moms

    moms, r = _spec_moments(c_ref[0], z)
    rs_sc[pl.ds(s, 1), :] = r.reshape(1, N)
    moms_sc[pl.ds(s, 1), :] = moms

    @pl.when(s == S_SRC - 1)
    def _():
        mm = moms_sc[...]
        m0m = mm[4:5, :]
        d = jnp.sqrt(jnp.sum((mm[:4, :] - m0m) ** 2, axis=1, keepdims=True))
        nd = -d / TAU
        ex = jnp.exp(nd - jnp.max(nd))
        rho = ex / jnp.sum(ex)
        rho_ref[...] = rho.reshape(1, S_SRC)
        den = jnp.sum(rs_sc[...] * rho, axis=0) + EPS
        den_ref[...] = den.reshape(N, 1)


def _k4(m0, cache, interpret=False):
    return pl.pallas_call(
        _k4_body,
        grid=(S_SRC,),
        in_specs=[
            pl.BlockSpec((1, N, N), lambda s: (s, 0, 0)),
            pl.BlockSpec((N, N), lambda s: (0, 0)),
            pl.BlockSpec((N, SPEC_RANK), lambda s: (0, 0)),
        ],
        out_specs=[
            pl.BlockSpec((1, S_SRC), lambda s: (0, 0)),
            pl.BlockSpec((N, 1), lambda s: (0, 0)),
        ],
        out_shape=(jax.ShapeDtypeStruct((1, S_SRC), jnp.float32),
                   jax.ShapeDtypeStruct((N, 1), jnp.float32)),
        scratch_shapes=[pltpu.VMEM((8, SPEC_MOM), jnp.float32),
                        pltpu.VMEM((S_SRC, N), jnp.float32)],
        interpret=interpret,
    )(cache, m0, jnp.asarray(_Z))


def _k5_body(m_ref, w0_ref, h_ref):
    m = m_ref[...]
    scale = jnp.sqrt(jnp.mean(m) / ROLE_DIM + EPS)
    w = w0_ref[...] * scale

    def it(_, w):
        num = jnp.dot(m, w, preferred_element_type=jnp.float32)
        wtw = lax.dot_general(w, w, (((0,), (0,)), ((), ())),
                              preferred_element_type=jnp.float32)
        den = jnp.dot(w, wtw, preferred_element_type=jnp.float32) + EPS
        return jnp.maximum(w * (num / den), 0.0)

    w = lax.fori_loop(0, ROLE_ITERS, it, w)
    nrm = jnp.sqrt(jnp.sum(w * w, axis=1, keepdims=True))
    h = w / (nrm + EPS)
    h_ref[...] = jnp.concatenate([h, jnp.zeros((N, 8), jnp.float32)], axis=1)


def _k5(m0, interpret=False):
    return pl.pallas_call(
        _k5_body,
        out_shape=jax.ShapeDtypeStruct((N, 16), jnp.float32),
        interpret=interpret,
    )(m0, jnp.asarray(_W0))


_NCHUNK = CH // 128


def _sc_gather_body(idxf, alr, cachef, table, vlr, vc, xj,
                    fidx_v, jidx_v, cidx_v, val_v, vc_v, xj_v, sem):
    wid = lax.axis_index("s") * 2 + lax.axis_index("c")
    pbase = wid * CH

    for kk in range(_NCHUNK):
        pltpu.sync_copy(idxf.at[pl.ds(pbase + kk * 128, 128)], fidx_v.at[kk])

    for v in range(CH // 16):
        r, c = divmod(v, 8)
        f = fidx_v[r, pl.ds(c * 16, 16)]
        jidx_v[r, pl.ds(c * 16, 16)] = lax.bitwise_and(f, N - 1)
        for s in range(S_SRC):
            cidx_v[s * _NCHUNK + r, pl.ds(c * 16, 16)] = f + s * (N * N)

    cps = []
    for kk in range(_NCHUNK):
        cps.append(pltpu.async_copy(table.at[jidx_v.at[kk]],
                                    xj_v.at[pl.ds(kk * 128, 128)], sem))
        cps.append(pltpu.async_copy(alr.at[fidx_v.at[kk]],
                                    val_v.at[pl.ds(kk * 128, 128)], sem))
        for s in range(S_SRC):
            cps.append(pltpu.async_copy(cachef.at[cidx_v.at[s * _NCHUNK + kk]],
                                        vc_v.at[s, pl.ds(kk * 128, 128)], sem))
    for cp in cps:
        cp.wait()

    pltpu.sync_copy(xj_v, xj.at[pl.ds(pbase, CH)])
    pltpu.sync_copy(val_v, vlr.at[pl.ds(pbase, CH)])
    for s in range(S_SRC):
        pltpu.sync_copy(vc_v.at[s], vc.at[pl.ds(s * E + pbase, CH)])


def _sc_gather(idx_flat, alr_flat, cache_flat, table):
    mesh = plsc.VectorSubcoreMesh(core_axis_name="c", subcore_axis_name="s")
    fn = pl.kernel(
        _sc_gather_body, mesh=mesh,
        out_type=(jax.ShapeDtypeStruct((E,), jnp.float32),
                  jax.ShapeDtypeStruct((S_SRC * E,), jnp.float32),
                  jax.ShapeDtypeStruct((E, TBL_W), jnp.float32)),
        scratch_types=[
            pltpu.VMEM((_NCHUNK, 128), jnp.int32),
            pltpu.VMEM((_NCHUNK, 128), jnp.int32),
            pltpu.VMEM((S_SRC * _NCHUNK, 128), jnp.int32),
            pltpu.VMEM((CH,), jnp.float32),
            pltpu.VMEM((S_SRC, CH), jnp.float32),
            pltpu.VMEM((CH, TBL_W), jnp.float32),
            pltpu.SemaphoreType.DMA,
        ],
    )
    return fn(idx_flat, alr_flat, cache_flat, table)


def _k6_body(s_ref, h_ref, xj_ref, cv_ref, rho_ref, den_ref,
             w1_ref, b1_ref, w2_ref, b2_ref, sr_ref, res_ref):
    xj = xj_ref[...]
    xjs = xj[:, :T]
    hj = xj[:, T:T + ROLE_DIM]
    sb = s_ref[...]
    hb = h_ref[...][:, :ROLE_DIM]

    rv = jnp.sum(hb * hj, axis=1, keepdims=True)
    cv = cv_ref[...]
    rho = jnp.reshape(rho_ref[...], (S_SRC, 1))
    num = jnp.sum(cv * rho, axis=0)[:, None]
    sr = rv * num / den_ref[...]
    sr_ref[...] = sr.reshape(1, 1, N)

    feats = []
    for l in range(8):
        f = jnp.sum(sb[:, l:] * xjs[:, :T - l], axis=1, keepdims=True) / (T - l)
        feats.append(f)
    fe = jnp.concatenate(feats, axis=1)
    h1 = jnp.maximum(jnp.dot(fe, w1_ref[...], preferred_element_type=jnp.float32)
                     + b1_ref[...], 0.0)
    res = jnp.sum(h1 * w2_ref[...], axis=1, keepdims=True) + b2_ref[...]
    res_ref[...] = res.reshape(1, 1, N)


def _k6(s, h16, xj, cv2, rho, den, w1, b1r, w2r, b2r, interpret=False):
    return pl.pallas_call(
        _k6_body,
        grid=(K,),
        in_specs=[
            pl.BlockSpec((N, T), lambda k: (0, 0)),
            pl.BlockSpec((N, 16), lambda k: (0, 0)),
            pl.BlockSpec((N, TBL_W), lambda k: (k, 0)),
            pl.BlockSpec((S_SRC, N), lambda k: (0, k)),
            pl.BlockSpec((1, S_SRC), lambda k: (0, 0)),
            pl.BlockSpec((N, 1), lambda k: (0, 0)),
            pl.BlockSpec((8, 32), lambda k: (0, 0)),
            pl.BlockSpec((1, 32), lambda k: (0, 0)),
            pl.BlockSpec((1, 32), lambda k: (0, 0)),
            pl.BlockSpec((1, 1), lambda k: (0, 0)),
        ],
        out_specs=[
            pl.BlockSpec((1, 1, N), lambda k: (k, 0, 0)),
            pl.BlockSpec((1, 1, N), lambda k: (k, 0, 0)),
        ],
        out_shape=(jax.ShapeDtypeStruct((K, 1, N), jnp.float32),
                   jax.ShapeDtypeStruct((K, 1, N), jnp.float32)),
        interpret=interpret,
    )(s, h16, xj, cv2, rho, den, w1, b1r, w2r, b2r)


def _k7_body(lr_ref, sr_ref, res_ref, p_ref):
    def nzs(v):
        mu = jnp.mean(v)
        sd = jnp.sqrt(jnp.mean((v - mu) ** 2))
        return (v - mu) / (sd + EPS)

    sf = (ALPHA * nzs(lr_ref[...]) + BETA * nzs(sr_ref[...])
          + DELTA * nzs(res_ref[...]))
    sig = 1.0 / (1.0 + jnp.exp(-sf))
    mx = jnp.max(sig, axis=0, keepdims=True)
    e = jnp.exp(sig - mx)
    p_ref[...] = e / jnp.sum(e, axis=0, keepdims=True)


def _k7(lr, sr, res, interpret=False):
    return pl.pallas_call(
        _k7_body,
        out_shape=jax.ShapeDtypeStruct((K, N), jnp.float32),
        interpret=interpret,
    )(lr, sr, res)


def _sc_scatter_body(idxf, pvals, zeros, out, fidx_v, buf_v, p_v, sem):
    wid = lax.axis_index("s") * 2 + lax.axis_index("c")
    pbase = wid * CH
    rbase = wid * RW
    pltpu.sync_copy(zeros, buf_v)
    pltpu.sync_copy(buf_v, out.at[pl.ds(rbase * N, RW * N)])
    for kk in range(_NCHUNK):
        pltpu.sync_copy(idxf.at[pl.ds(pbase + kk * 128, 128)], fidx_v.at[kk])
    pltpu.sync_copy(pvals.at[pl.ds(pbase, CH)], p_v)
    cps = [pltpu.async_copy(p_v.at[pl.ds(kk * 128, 128)],
                            out.at[fidx_v.at[kk]], sem)
           for kk in range(_NCHUNK)]
    for cp in cps:
        cp.wait()


def _sc_scatter(idx_flat, p_flat, zeros):
    mesh = plsc.VectorSubcoreMesh(core_axis_name="c", subcore_axis_name="s")
    fn = pl.kernel(
        _sc_scatter_body, mesh=mesh,
        out_type=jax.ShapeDtypeStruct((N * N,), jnp.float32),
        scratch_types=[
            pltpu.VMEM((_NCHUNK, 128), jnp.int32),
            pltpu.VMEM((RW * N,), jnp.float32),
            pltpu.VMEM((CH,), jnp.float32),
            pltpu.SemaphoreType.DMA,
        ],
    )
    return fn(idx_flat, p_flat, zeros)


def kernel(x_target, source_structure_cache, W1, b1, W2, b2):
    xt = x_target.transpose(0, 3, 1, 2).reshape(8, N, T)
    s, m0 = _k1(xt)
    idx = _k2(m0)
    alr = _k3(m0)
    rho, den = _k4(m0, source_structure_cache)
    h16 = _k5(m0)

    idx_flat = idx.reshape(E)
    idx_kmaj = jnp.transpose(idx).reshape(E)
    table = jnp.concatenate(
        [s, h16, jnp.zeros((N, TBL_W - T - 16), jnp.float32)], axis=1)
    vlr, vc, xj = _sc_gather(idx_kmaj, alr.reshape(N * N),
                             source_structure_cache.reshape(S_SRC * N * N), table)

    sr, res = _k6(s, h16, xj, vc.reshape(S_SRC, E), rho, den,
                  W1, b1.reshape(1, 32), W2.reshape(1, 32), b2.reshape(1, 1))
    p = _k7(vlr.reshape(K, N), sr.reshape(K, N), res.reshape(K, N))

    zeros = jnp.zeros((RW * N,), jnp.float32)
    out = _sc_scatter(idx_flat, jnp.transpose(p).reshape(E), zeros)
    return out.reshape(N, N)
